# baseline (speedup 1.0000x reference)
import functools

import jax
import jax.numpy as jnp
from jax import lax
from jax.experimental import pallas as pl
from jax.experimental.pallas import tpu as pltpu
from jax.experimental.pallas import tpu_sc as plsc

E = 16
K = 2
D = 2048
N = 4 * 4096
R = 512
NT = N // R
L = 16
G = R // L


def _tc_router(x_ref, w_ref, lt_ref, loss_ref, acc_ref):
    i = pl.program_id(0)
    lt = lax.dot_general(
        w_ref[...], x_ref[...],
        dimension_numbers=(((1,), (1,)), ((), ())),
        preferred_element_type=jnp.float32,
    )
    lt_ref[...] = lt[None]
    m = jnp.max(lt, axis=0, keepdims=True)
    p = jnp.exp(lt - m)
    s = jnp.sum(p, axis=0, keepdims=True)
    part = jnp.sum(p / s, axis=1, keepdims=True)

    @pl.when(i == 0)
    def _():
        acc_ref[...] = jnp.zeros_like(acc_ref)

    acc_ref[...] += part

    @pl.when(i == pl.num_programs(0) - 1)
    def _():
        usage = acc_ref[...] * (1.0 / N)
        loss_ref[...] = E * jnp.sum(usage * usage, axis=(0, 1), keepdims=True)


def _tc_call(xf, w):
    return pl.pallas_call(
        _tc_router,
        grid=(NT,),
        in_specs=[
            pl.BlockSpec((R, D), lambda i: (i, 0)),
            pl.BlockSpec((E, D), lambda i: (0, 0)),
        ],
        out_specs=[
            pl.BlockSpec((1, E, R), lambda i: (i, 0, 0)),
            pl.BlockSpec((1, 1), lambda i: (0, 0)),
        ],
        out_shape=[
            jax.ShapeDtypeStruct((NT, E, R), jnp.float32),
            jax.ShapeDtypeStruct((1, 1), jnp.float32),
        ],
        scratch_shapes=[pltpu.VMEM((E, 1), jnp.float32)],
    )(xf, w)


def _sc_router(lt_hbm, w_out, i_out, lt_v, w_v, i_v):
    nc = 2
    wid = lax.axis_index("s") * nc + lax.axis_index("c")
    pltpu.sync_copy(lt_hbm.at[wid], lt_v)

    def group(g, _):
        sl = pl.ds(g * L, L)
        m1 = lt_v[0, sl]
        i1 = jnp.zeros((L,), jnp.int32)
        m2 = jnp.full((L,), -3.0e38, jnp.float32)
        i2 = jnp.zeros((L,), jnp.int32)
        for e in range(1, E):
            le = lt_v[e, sl]
            es = jnp.full((L,), e, jnp.int32)
            gt1 = le > m1
            gt2 = le > m2
            n_m2 = jnp.where(gt1, m1, jnp.where(gt2, le, m2))
            n_i2 = jnp.where(gt1, i1, jnp.where(gt2, es, i2))
            m1 = jnp.where(gt1, le, m1)
            i1 = jnp.where(gt1, es, i1)
            m2 = n_m2
            i2 = n_i2
        e21 = jnp.exp(m2 - m1)
        den = 1.0 + e21
        w1 = 1.0 / den
        w2 = e21 / den
        w_v[0, sl] = w1
        w_v[1, sl] = w2
        i_v[0, sl] = i1
        i_v[1, sl] = i2
        return _

    lax.fori_loop(0, G, group, 0)

    base = wid * R
    pltpu.sync_copy(w_v, w_out.at[:, pl.ds(base, R)])
    pltpu.sync_copy(i_v, i_out.at[:, pl.ds(base, R)])


@functools.lru_cache(maxsize=1)
def _sc_call():
    return pl.kernel(
        _sc_router,
        mesh=plsc.VectorSubcoreMesh(core_axis_name="c", subcore_axis_name="s"),
        out_type=[
            jax.ShapeDtypeStruct((K, N), jnp.float32),
            jax.ShapeDtypeStruct((K, N), jnp.int32),
        ],
        scratch_types=[
            pltpu.VMEM((E, R), jnp.float32),
            pltpu.VMEM((K, R), jnp.float32),
            pltpu.VMEM((K, R), jnp.int32),
        ],
    )


def kernel(x, W):
    b, t, d = x.shape
    xf = x.reshape(b * t, d)
    lt, loss = _tc_call(xf, W)
    ws, idx = _sc_call()(lt)
    return (
        ws.T.reshape(b, t, K),
        idx.T.reshape(b, t, K),
        loss[0, 0],
    )


# speedup vs baseline: 1.1677x; 1.1677x over previous
import functools

import jax
import jax.numpy as jnp
from jax import lax
from jax.experimental import pallas as pl
from jax.experimental.pallas import tpu as pltpu
from jax.experimental.pallas import tpu_sc as plsc

E = 16
K = 2
D = 2048
N = 4 * 4096
R = 1024
NT = N // R
NW = 32
RS = N // NW
L = 16
G = RS // L


def _tc_router(x_ref, w_ref, lt_ref, loss_ref, acc_ref):
    i = pl.program_id(0)
    lt = lax.dot_general(
        w_ref[...], x_ref[...],
        dimension_numbers=(((1,), (1,)), ((), ())),
        preferred_element_type=jnp.float32,
    )
    lt_ref[...] = lt
    m = jnp.max(lt, axis=0, keepdims=True)
    p = jnp.exp(lt - m)
    s = jnp.sum(p, axis=0, keepdims=True)
    part = jnp.sum(p / s, axis=1, keepdims=True)

    @pl.when(i == 0)
    def _():
        acc_ref[...] = jnp.zeros_like(acc_ref)

    acc_ref[...] += part

    @pl.when(i == pl.num_programs(0) - 1)
    def _():
        usage = acc_ref[...] * (1.0 / N)
        loss_ref[...] = E * jnp.sum(usage * usage, axis=(0, 1), keepdims=True)


def _tc_call(xf, w):
    return pl.pallas_call(
        _tc_router,
        grid=(NT,),
        in_specs=[
            pl.BlockSpec((R, D), lambda i: (i, 0)),
            pl.BlockSpec((E, D), lambda i: (0, 0)),
        ],
        out_specs=[
            pl.BlockSpec((E, R), lambda i: (0, i)),
            pl.BlockSpec((1, 1), lambda i: (0, 0)),
        ],
        out_shape=[
            jax.ShapeDtypeStruct((E, N), jnp.float32),
            jax.ShapeDtypeStruct((1, 1), jnp.float32),
        ],
        scratch_shapes=[pltpu.VMEM((E, 1), jnp.float32)],
    )(xf, w)


def _sc_router(lt_hbm, w_out, i_out, lt_v, w_v, i_v):
    nc = 2
    wid = lax.axis_index("s") * nc + lax.axis_index("c")
    base = wid * RS
    pltpu.sync_copy(lt_hbm.at[:, pl.ds(base, RS)], lt_v)

    def group(g, _):
        sl = pl.ds(g * L, L)
        m1 = lt_v[0, sl]
        i1 = jnp.zeros((L,), jnp.int32)
        m2 = jnp.full((L,), -3.0e38, jnp.float32)
        i2 = jnp.zeros((L,), jnp.int32)
        for e in range(1, E):
            le = lt_v[e, sl]
            es = jnp.full((L,), e, jnp.int32)
            gt1 = le > m1
            gt2 = le > m2
            n_m2 = jnp.where(gt1, m1, jnp.where(gt2, le, m2))
            n_i2 = jnp.where(gt1, i1, jnp.where(gt2, es, i2))
            m1 = jnp.where(gt1, le, m1)
            i1 = jnp.where(gt1, es, i1)
            m2 = n_m2
            i2 = n_i2
        e21 = jnp.exp(m2 - m1)
        den = 1.0 + e21
        w1 = 1.0 / den
        w2 = e21 / den
        w_v[0, sl] = w1
        w_v[1, sl] = w2
        i_v[0, sl] = i1
        i_v[1, sl] = i2
        return _

    lax.fori_loop(0, G, group, 0)

    pltpu.sync_copy(w_v, w_out.at[:, pl.ds(base, RS)])
    pltpu.sync_copy(i_v, i_out.at[:, pl.ds(base, RS)])


@functools.lru_cache(maxsize=1)
def _sc_call():
    return pl.kernel(
        _sc_router,
        mesh=plsc.VectorSubcoreMesh(core_axis_name="c", subcore_axis_name="s"),
        out_type=[
            jax.ShapeDtypeStruct((K, N), jnp.float32),
            jax.ShapeDtypeStruct((K, N), jnp.int32),
        ],
        scratch_types=[
            pltpu.VMEM((E, RS), jnp.float32),
            pltpu.VMEM((K, RS), jnp.float32),
            pltpu.VMEM((K, RS), jnp.int32),
        ],
    )


def kernel(x, W):
    b, t, d = x.shape
    xf = x.reshape(b * t, d)
    lt, loss = _tc_call(xf, W)
    ws, idx = _sc_call()(lt)
    return (
        ws.T.reshape(b, t, K),
        idx.T.reshape(b, t, K),
        loss[0, 0],
    )
